# baseline (speedup 1.0000x reference)
import functools

import jax
import jax.numpy as jnp
from jax import lax
from jax.experimental import pallas as pl
from jax.experimental.pallas import tpu as pltpu
from jax.experimental.pallas import tpu_sc as plsc

_N = 10000
_D = 128
_E = 320000

_NC = 2
_NS = 16
_NW = _NC * _NS
_L = 16
_C = 128
_NCHUNK = _E // _C
_CH = _NCHUNK // _NW
_XTRA = _NCHUNK - _NW * _CH
_RPS = 632
_N_PAD = _NS * _RPS


def _spmm_body(x_hbm, src_hbm, dst_hbm, val_hbm, out_hbm,
               s0, s1, s2, d0, d1, v0, v1, rows0, rows1, rows2, acc_sh,
               sem_src, sem_dst, sem_val,
               sem_g0, sem_g1, sem_g2, sem_s0, sem_s1, sem_s2):
    cid = lax.axis_index("c")
    sid = lax.axis_index("s")
    wid = sid * _NC + cid
    ebase = wid * _CH * _C

    bufs = (rows0, rows1, rows2)
    srcs = (s0, s1, s2)
    dsts = (d0, d1)
    vals = (v0, v1)
    gsems = (sem_g0, sem_g1, sem_g2)
    ssems = (sem_s0, sem_s1, sem_s2)

    def _zrow(i, c):
        for j in range(_D // _L):
            rows0[i, pl.ds(j * _L, _L)] = jnp.zeros((_L,), jnp.float32)
        return c

    lax.fori_loop(0, _C, _zrow, 0)
    zbase = sid * _RPS
    for k in range(_RPS // _C):
        pltpu.sync_copy(rows0, acc_sh.at[pl.ds(zbase + k * _C, _C)])
    _rem = _RPS % _C
    if _rem:
        pltpu.sync_copy(rows0.at[pl.ds(0, _rem)],
                        acc_sh.at[pl.ds(zbase + (_RPS // _C) * _C, _rem)])

    pltpu.sync_copy(src_hbm.at[pl.ds(ebase + 0 * _C, _C)], s0)
    pltpu.sync_copy(src_hbm.at[pl.ds(ebase + 1 * _C, _C)], s1)
    pltpu.sync_copy(dst_hbm.at[pl.ds(ebase + 0 * _C, _C)], d0)
    pltpu.sync_copy(val_hbm.at[pl.ds(ebase + 0 * _C, _C)], v0)
    pltpu.async_copy(src_hbm.at[pl.ds(ebase + 2 * _C, _C)], s2, sem_src)

    plsc.subcore_barrier()

    def _gissue(g, b):
        pltpu.async_copy(x_hbm.at[srcs[b]], bufs[b], gsems[b])

    def _gwait(g, b):
        pltpu.make_async_copy(x_hbm.at[srcs[b]], bufs[b], gsems[b]).wait()

    def _sissue(g, b, q):
        pltpu.async_copy(bufs[b], acc_sh.at[dsts[q]], ssems[b], add=True)

    def _swait(g, b, q):
        pltpu.make_async_copy(bufs[b], acc_sh.at[dsts[q]], ssems[b]).wait()

    def _src_wait(g, b):
        pltpu.make_async_copy(src_hbm.at[pl.ds(ebase + g * _C, _C)],
                              srcs[b], sem_src).wait()

    def _dst_wait(g, q):
        pltpu.make_async_copy(dst_hbm.at[pl.ds(ebase + g * _C, _C)],
                              dsts[q], sem_dst).wait()

    def _val_wait(g, q):
        pltpu.make_async_copy(val_hbm.at[pl.ds(ebase + g * _C, _C)],
                              vals[q], sem_val).wait()

    def _scale(b, q):
        rows_b = bufs[b]
        val_q = vals[q]

        def _grp(gr, cc):
            vals16 = val_q[pl.ds(gr * _L, _L)]
            for l in range(_L):
                vv = jnp.full((_L,), vals16[l], jnp.float32)
                e = gr * _L + l
                for j in range(_D // _L):
                    sl = pl.ds(j * _L, _L)
                    rows_b[e, sl] = rows_b[e, sl] * vv
            return cc

        lax.fori_loop(0, _C // _L, _grp, 0)

    def _src_issue(g, b):
        pltpu.async_copy(src_hbm.at[pl.ds(ebase + g * _C, _C)],
                         srcs[b], sem_src)

    def _dv_issue(g, q):
        pltpu.async_copy(dst_hbm.at[pl.ds(ebase + g * _C, _C)],
                         dsts[q], sem_dst)
        pltpu.async_copy(val_hbm.at[pl.ds(ebase + g * _C, _C)],
                         vals[q], sem_val)

    def _body(g, b, q, first=False, n_ahead=3):
        _gwait(g, b)
        if not first:
            _swait(g - 1, (b + 2) % 3, 1 - q)
        if n_ahead >= 2:
            _src_wait(g + 2, (b + 2) % 3)
            _gissue(g + 2, (b + 2) % 3)
        if not first:
            _val_wait(g, q)
        if n_ahead >= 3:
            _src_issue(g + 3, b)
        _scale(b, q)
        if not first:
            _dst_wait(g, q)
        _sissue(g, b, q)
        if n_ahead >= 1:
            _dv_issue(g + 1, 1 - q)

    _gissue(0, 0)
    _gissue(1, 1)
    _body(0, 0, 0, first=True)

    def _main(i, c):
        g0 = 1 + i * 6
        for k in range(6):
            _body(g0 + k, (1 + k) % 3, (1 + k) % 2)
        return c

    lax.fori_loop(0, 12, _main, 0)

    for g in range(73, _CH):
        _body(g, g % 3, g % 2, n_ahead=min(3, _CH - 1 - g))
    _swait(_CH - 1, (_CH - 1) % 3, (_CH - 1) % 2)

    @pl.when(wid < _XTRA)
    def _extra():
        off = (_NW * _CH + wid) * _C
        pltpu.sync_copy(src_hbm.at[pl.ds(off, _C)], s0)
        pltpu.sync_copy(dst_hbm.at[pl.ds(off, _C)], d0)
        pltpu.sync_copy(val_hbm.at[pl.ds(off, _C)], v0)
        pltpu.async_copy(x_hbm.at[s0], rows0, sem_g0).wait()
        _scale(0, 0)
        pltpu.sync_copy(rows0, acc_sh.at[d0], add=True)

    plsc.subcore_barrier()
    pltpu.sync_copy(acc_sh.at[pl.ds(zbase, _RPS)],
                    out_hbm.at[cid, pl.ds(zbase, _RPS)])


_spmm = functools.partial(
    pl.kernel,
    out_type=jax.ShapeDtypeStruct((_NC, _N_PAD, _D), jnp.float32),
    mesh=plsc.VectorSubcoreMesh(core_axis_name="c", subcore_axis_name="s",
                                num_cores=_NC, num_subcores=_NS),
    scratch_types=[
        pltpu.VMEM((_C,), jnp.int32),
        pltpu.VMEM((_C,), jnp.int32),
        pltpu.VMEM((_C,), jnp.int32),
        pltpu.VMEM((_C,), jnp.int32),
        pltpu.VMEM((_C,), jnp.int32),
        pltpu.VMEM((_C,), jnp.float32),
        pltpu.VMEM((_C,), jnp.float32),
        pltpu.VMEM((_C, _D), jnp.float32),
        pltpu.VMEM((_C, _D), jnp.float32),
        pltpu.VMEM((_C, _D), jnp.float32),
        pltpu.VMEM_SHARED((_N_PAD, _D), jnp.float32),
        pltpu.SemaphoreType.DMA,
        pltpu.SemaphoreType.DMA,
        pltpu.SemaphoreType.DMA,
        pltpu.SemaphoreType.DMA,
        pltpu.SemaphoreType.DMA,
        pltpu.SemaphoreType.DMA,
        pltpu.SemaphoreType.DMA,
        pltpu.SemaphoreType.DMA,
        pltpu.SemaphoreType.DMA,
    ],
)(_spmm_body)


_BLK = 1000


def _dense_body(p0, p1, w, b, o, *, relu):
    s = p0[0] + p1[0]
    y = jnp.dot(s, w[...], preferred_element_type=jnp.float32) + b[...]
    o[...] = jnp.maximum(y, 0.0) if relu else y


def _dense(p, w, b, relu):
    return pl.pallas_call(
        functools.partial(_dense_body, relu=relu),
        out_shape=jax.ShapeDtypeStruct((_N, _D), jnp.float32),
        grid=(_N // _BLK,),
        in_specs=[
            pl.BlockSpec((1, _BLK, _D), lambda i: (0, i, 0)),
            pl.BlockSpec((1, _BLK, _D), lambda i: (1, i, 0)),
            pl.BlockSpec((_D, _D), lambda i: (0, 0)),
            pl.BlockSpec((1, _D), lambda i: (0, 0)),
        ],
        out_specs=pl.BlockSpec((_BLK, _D), lambda i: (i, 0)),
    )(p, p, w, b.reshape(1, _D))


def kernel(x, edge_index, edge_values, W1, b1, W2, b2):
    dst = edge_index[0]
    src = edge_index[1]

    p1 = _spmm(x, src, dst, edge_values)
    h = _dense(p1, W1, b1, relu=True)
    p2 = _spmm(h, src, dst, edge_values)
    out = _dense(p2, W2, b2, relu=False)
    return out


# speedup vs baseline: 1.0216x; 1.0216x over previous
import functools

import jax
import jax.numpy as jnp
from jax import lax
from jax.experimental import pallas as pl
from jax.experimental.pallas import tpu as pltpu
from jax.experimental.pallas import tpu_sc as plsc

_N = 10000
_D = 128
_E = 320000

_NC = 2
_NS = 16
_NW = _NC * _NS
_L = 16
_C = 128
_NCHUNK = _E // _C
_CH = _NCHUNK // _NW
_XTRA = _NCHUNK - _NW * _CH
_RPS = 632
_N_PAD = _NS * _RPS


def _spmm_body(x_hbm, src_hbm, dst_hbm, val_hbm, out_hbm,
               s0, s1, s2, d0, d1, v0, v1, rows0, rows1, rows2, acc_sh,
               sem_src, sem_dst, sem_val,
               sem_g0, sem_g1, sem_g2, sem_s0, sem_s1, sem_s2):
    cid = lax.axis_index("c")
    sid = lax.axis_index("s")
    wid = sid * _NC + cid
    ebase = wid * _CH * _C

    bufs = (rows0, rows1, rows2)
    srcs = (s0, s1, s2)
    dsts = (d0, d1)
    vals = (v0, v1)
    gsems = (sem_g0, sem_g1, sem_g2)
    ssems = (sem_s0, sem_s1, sem_s2)

    def _zrow(i, c):
        for j in range(_D // _L):
            rows0[i, pl.ds(j * _L, _L)] = jnp.zeros((_L,), jnp.float32)
        return c

    lax.fori_loop(0, _C, _zrow, 0)
    zbase = sid * _RPS
    for k in range(_RPS // _C):
        pltpu.sync_copy(rows0, acc_sh.at[pl.ds(zbase + k * _C, _C)])
    _rem = _RPS % _C
    if _rem:
        pltpu.sync_copy(rows0.at[pl.ds(0, _rem)],
                        acc_sh.at[pl.ds(zbase + (_RPS // _C) * _C, _rem)])

    pltpu.sync_copy(src_hbm.at[pl.ds(ebase + 0 * _C, _C)], s0)
    pltpu.sync_copy(src_hbm.at[pl.ds(ebase + 1 * _C, _C)], s1)
    pltpu.sync_copy(dst_hbm.at[pl.ds(ebase + 0 * _C, _C)], d0)
    pltpu.sync_copy(val_hbm.at[pl.ds(ebase + 0 * _C, _C)], v0)
    pltpu.async_copy(src_hbm.at[pl.ds(ebase + 2 * _C, _C)], s2, sem_src)

    plsc.subcore_barrier()

    def _gissue(g, b):
        pltpu.async_copy(x_hbm.at[srcs[b]], bufs[b], gsems[b])

    def _gwait(g, b):
        pltpu.make_async_copy(x_hbm.at[srcs[b]], bufs[b], gsems[b]).wait()

    def _sissue(g, b, q):
        pltpu.async_copy(bufs[b], acc_sh.at[dsts[q]], ssems[b], add=True)

    def _swait(g, b, q):
        pltpu.make_async_copy(bufs[b], acc_sh.at[dsts[q]], ssems[b]).wait()

    def _src_wait(g, b):
        pltpu.make_async_copy(src_hbm.at[pl.ds(ebase + g * _C, _C)],
                              srcs[b], sem_src).wait()

    def _dst_wait(g, q):
        pltpu.make_async_copy(dst_hbm.at[pl.ds(ebase + g * _C, _C)],
                              dsts[q], sem_dst).wait()

    def _val_wait(g, q):
        pltpu.make_async_copy(val_hbm.at[pl.ds(ebase + g * _C, _C)],
                              vals[q], sem_val).wait()

    def _scale(b, q):
        rows_b = bufs[b]
        val_q = vals[q]

        def _grp(gr, cc):
            vals16 = val_q[pl.ds(gr * _L, _L)]
            for l in range(_L):
                vv = jnp.full((_L,), vals16[l], jnp.float32)
                e = gr * _L + l
                for j in range(_D // _L):
                    sl = pl.ds(j * _L, _L)
                    rows_b[e, sl] = rows_b[e, sl] * vv
            return cc

        lax.fori_loop(0, _C // _L, _grp, 0)

    def _src_issue(g, b):
        pltpu.async_copy(src_hbm.at[pl.ds(ebase + g * _C, _C)],
                         srcs[b], sem_src)

    def _dv_issue(g, q):
        pltpu.async_copy(dst_hbm.at[pl.ds(ebase + g * _C, _C)],
                         dsts[q], sem_dst)
        pltpu.async_copy(val_hbm.at[pl.ds(ebase + g * _C, _C)],
                         vals[q], sem_val)

    def _body(g, b, q, first=False, n_ahead=3):
        _gwait(g, b)
        if not first:
            _val_wait(g, q)
        if n_ahead >= 3:
            _src_issue(g + 3, b)
        _scale(b, q)
        if not first:
            _dst_wait(g, q)
        _sissue(g, b, q)
        if not first:
            _swait(g - 1, (b + 2) % 3, 1 - q)
        if n_ahead >= 2:
            _src_wait(g + 2, (b + 2) % 3)
            _gissue(g + 2, (b + 2) % 3)
        if n_ahead >= 1:
            _dv_issue(g + 1, 1 - q)

    _gissue(0, 0)
    _gissue(1, 1)
    _body(0, 0, 0, first=True)

    def _main(i, c):
        g0 = 1 + i * 6
        for k in range(6):
            _body(g0 + k, (1 + k) % 3, (1 + k) % 2)
        return c

    lax.fori_loop(0, 12, _main, 0)

    for g in range(73, _CH):
        _body(g, g % 3, g % 2, n_ahead=min(3, _CH - 1 - g))
    _swait(_CH - 1, (_CH - 1) % 3, (_CH - 1) % 2)

    @pl.when(wid < _XTRA)
    def _extra():
        off = (_NW * _CH + wid) * _C
        pltpu.sync_copy(src_hbm.at[pl.ds(off, _C)], s0)
        pltpu.sync_copy(dst_hbm.at[pl.ds(off, _C)], d0)
        pltpu.sync_copy(val_hbm.at[pl.ds(off, _C)], v0)
        pltpu.async_copy(x_hbm.at[s0], rows0, sem_g0).wait()
        _scale(0, 0)
        pltpu.sync_copy(rows0, acc_sh.at[d0], add=True)

    plsc.subcore_barrier()
    pltpu.sync_copy(acc_sh.at[pl.ds(zbase, _RPS)],
                    out_hbm.at[cid, pl.ds(zbase, _RPS)])


_spmm = functools.partial(
    pl.kernel,
    out_type=jax.ShapeDtypeStruct((_NC, _N_PAD, _D), jnp.float32),
    mesh=plsc.VectorSubcoreMesh(core_axis_name="c", subcore_axis_name="s",
                                num_cores=_NC, num_subcores=_NS),
    scratch_types=[
        pltpu.VMEM((_C,), jnp.int32),
        pltpu.VMEM((_C,), jnp.int32),
        pltpu.VMEM((_C,), jnp.int32),
        pltpu.VMEM((_C,), jnp.int32),
        pltpu.VMEM((_C,), jnp.int32),
        pltpu.VMEM((_C,), jnp.float32),
        pltpu.VMEM((_C,), jnp.float32),
        pltpu.VMEM((_C, _D), jnp.float32),
        pltpu.VMEM((_C, _D), jnp.float32),
        pltpu.VMEM((_C, _D), jnp.float32),
        pltpu.VMEM_SHARED((_N_PAD, _D), jnp.float32),
        pltpu.SemaphoreType.DMA,
        pltpu.SemaphoreType.DMA,
        pltpu.SemaphoreType.DMA,
        pltpu.SemaphoreType.DMA,
        pltpu.SemaphoreType.DMA,
        pltpu.SemaphoreType.DMA,
        pltpu.SemaphoreType.DMA,
        pltpu.SemaphoreType.DMA,
        pltpu.SemaphoreType.DMA,
    ],
)(_spmm_body)


_BLK = 1000


def _dense_body(p0, p1, w, b, o, *, relu):
    s = p0[0] + p1[0]
    y = jnp.dot(s, w[...], preferred_element_type=jnp.float32) + b[...]
    o[...] = jnp.maximum(y, 0.0) if relu else y


def _dense(p, w, b, relu):
    return pl.pallas_call(
        functools.partial(_dense_body, relu=relu),
        out_shape=jax.ShapeDtypeStruct((_N, _D), jnp.float32),
        grid=(_N // _BLK,),
        in_specs=[
            pl.BlockSpec((1, _BLK, _D), lambda i: (0, i, 0)),
            pl.BlockSpec((1, _BLK, _D), lambda i: (1, i, 0)),
            pl.BlockSpec((_D, _D), lambda i: (0, 0)),
            pl.BlockSpec((1, _D), lambda i: (0, 0)),
        ],
        out_specs=pl.BlockSpec((_BLK, _D), lambda i: (i, 0)),
    )(p, p, w, b.reshape(1, _D))


def kernel(x, edge_index, edge_values, W1, b1, W2, b2):
    dst = edge_index[0]
    src = edge_index[1]

    p1 = _spmm(x, src, dst, edge_values)
    h = _dense(p1, W1, b1, relu=True)
    p2 = _spmm(h, src, dst, edge_values)
    out = _dense(p2, W2, b2, relu=False)
    return out
